# baseline (speedup 1.0000x reference)
import functools

import jax
import jax.numpy as jnp
from jax import lax
from jax.experimental import pallas as pl
from jax.experimental.pallas import tpu as pltpu, tpu_sc as plsc

VOCAB = 1000000
EMB_DIM = 32
BATCH = 16384
HIST = 50

_info = plsc.get_sparse_core_info()
NC, NS = _info.num_cores, _info.num_subcores
NW = NC * NS
L = _info.num_lanes

IB = 128
NI = BATCH // IB
UNITS = HIST * NI
UNITS_W = UNITS // NW
UPB = 2
CHUNK = UPB * IB
NBLK = UNITS_W // UPB
NBUF = 5
NWAVE = NBLK // NBUF
TILE = EMB_DIM * IB
NCH = TILE // L


def _make_kernel():
    mesh = plsc.VectorSubcoreMesh(core_axis_name="c", subcore_axis_name="s")

    @functools.partial(
        pl.kernel,
        mesh=mesh,
        out_type=jax.ShapeDtypeStruct((HIST, EMB_DIM // 8, NI * 8 * IB),
                                      jnp.float32),
        scratch_types=[
            pltpu.VMEM((UNITS_W * IB,), jnp.int32),
            pltpu.VMEM((NBUF * CHUNK, EMB_DIM), jnp.float32),
            pltpu.VMEM((NBUF * UPB * TILE,), jnp.float32),
            [pltpu.SemaphoreType.DMA] * NBUF,
            [pltpu.SemaphoreType.DMA] * (NBUF * UPB),
            pltpu.SemaphoreType.DMA,
        ],
        compiler_params=pltpu.CompilerParams(
            use_tc_tiling_on_sc=False, needs_layout_passes=False
        ),
    )
    def gather_kernel(xt_hbm, table_hbm, out_hbm, idx_v, rows_v, tout_v,
                      gsems, osems, isem):
        wid = lax.axis_index("s") * NC + lax.axis_index("c")
        ubase = wid * UNITS_W
        pltpu.async_copy(
            xt_hbm.at[pl.ds(ubase * IB, UNITS_W * IB)], idx_v, isem
        ).wait()

        iota = lax.iota(jnp.int32, L)
        iota128 = iota * IB

        def gather_blk(g, b):
            pltpu.async_copy(
                table_hbm.at[idx_v.at[pl.ds(g * CHUNK, CHUNK)]],
                rows_v.at[pl.ds(b * CHUNK, CHUNK)],
                gsems[b],
            )

        def gather_blk_wait(g, b):
            pltpu.make_async_copy(
                table_hbm.at[idx_v.at[pl.ds(g * CHUNK, CHUNK)]],
                rows_v.at[pl.ds(b * CHUNK, CHUNK)],
                gsems[b],
            ).wait()

        def store_unit(g, b, uu):
            u = ubase + g * UPB + uu
            h = u // NI
            i1 = u % NI
            t = b * UPB + uu
            for e1 in range(EMB_DIM // 8):
                pltpu.async_copy(
                    tout_v.at[pl.ds(t * TILE + e1 * 1024, 1024)],
                    out_hbm.at[h, e1, pl.ds(i1 * 1024, 1024)],
                    osems[t],
                )

        def store_unit_wait(g, b, uu):
            u = ubase + g * UPB + uu
            h = u // NI
            i1 = u % NI
            t = b * UPB + uu
            for e1 in range(EMB_DIM // 8):
                pltpu.make_async_copy(
                    tout_v.at[pl.ds(t * TILE + e1 * 1024, 1024)],
                    out_hbm.at[h, e1, pl.ds(i1 * 1024, 1024)],
                    osems[t],
                ).wait()

        def transpose_blk(b):
            row0 = b * CHUNK
            t0 = b * UPB
            base_a = iota128 + t0 * TILE
            base_b = base_a + L * IB
            base_c = iota128 + (t0 + 1) * TILE
            base_d = base_c + L * IB

            def row_body(i2, carry):
                va = rows_v[row0 + i2, pl.ds(0, L)]
                vb = rows_v[row0 + i2, pl.ds(L, L)]
                vc = rows_v[row0 + IB + i2, pl.ds(0, L)]
                vd = rows_v[row0 + IB + i2, pl.ds(L, L)]
                plsc.store_scatter(tout_v, [base_a + i2], va)
                plsc.store_scatter(tout_v, [base_b + i2], vb)
                plsc.store_scatter(tout_v, [base_c + i2], vc)
                plsc.store_scatter(tout_v, [base_d + i2], vd)
                return carry

            lax.fori_loop(0, IB, row_body, 0, unroll=8)

        for b in range(NBUF):
            gather_blk(b, b)

        def wave(i, carry):
            for b in range(NBUF):
                g = i * NBUF + b
                gather_blk_wait(g, b)
                @pl.when(i > 0)
                def _():
                    for uu in range(UPB):
                        store_unit_wait(g - NBUF, b, uu)
                transpose_blk(b)
                for uu in range(UPB):
                    store_unit(g, b, uu)
                @pl.when(i < NWAVE - 1)
                def _():
                    gather_blk(g + NBUF, b)
            return carry

        lax.fori_loop(0, NWAVE, wave, 0)

        for b in range(NBUF):
            for uu in range(UPB):
                store_unit_wait(NBLK - NBUF + b, b, uu)

    return gather_kernel


_gather = _make_kernel()


def kernel(x, table):
    xt_flat = x.T.reshape(BATCH * HIST).astype(jnp.int32)
    out3 = _gather(xt_flat, table)
    out5 = out3.reshape(HIST, EMB_DIM // 8, NI, 8, IB)
    return out5.transpose(2, 4, 0, 1, 3).reshape(BATCH, HIST, EMB_DIM)


# speedup vs baseline: 1.0609x; 1.0609x over previous
import functools

import jax
import jax.numpy as jnp
from jax import lax
from jax.experimental import pallas as pl
from jax.experimental.pallas import tpu as pltpu, tpu_sc as plsc

VOCAB = 1000000
EMB_DIM = 32
BATCH = 16384
HIST = 50

_info = plsc.get_sparse_core_info()
NC, NS = _info.num_cores, _info.num_subcores
NW = NC * NS
L = _info.num_lanes

IB = 128
NI = BATCH // IB
UNITS = HIST * NI
UNITS_W = UNITS // NW
UPB = 2
CHUNK = UPB * IB
NBLK = UNITS_W // UPB
NBUF = 5
NWAVE = NBLK // NBUF
TILE = EMB_DIM * IB
NCH = TILE // L


def _make_kernel():
    mesh = plsc.VectorSubcoreMesh(core_axis_name="c", subcore_axis_name="s")

    @functools.partial(
        pl.kernel,
        mesh=mesh,
        out_type=jax.ShapeDtypeStruct((HIST, EMB_DIM // 8, NI * 8 * IB),
                                      jnp.float32),
        scratch_types=[
            pltpu.VMEM((UNITS_W * IB,), jnp.int32),
            pltpu.VMEM((NBUF * CHUNK, EMB_DIM), jnp.float32),
            pltpu.VMEM((NBUF * UPB * TILE,), jnp.float32),
            [pltpu.SemaphoreType.DMA] * NBUF,
            [pltpu.SemaphoreType.DMA] * (NBUF * UPB),
            pltpu.SemaphoreType.DMA,
        ],
        compiler_params=pltpu.CompilerParams(
            use_tc_tiling_on_sc=False, needs_layout_passes=False
        ),
    )
    def gather_kernel(xt_hbm, table_hbm, out_hbm, idx_v, rows_v, tout_v,
                      gsems, osems, isem):
        wid = lax.axis_index("s") * NC + lax.axis_index("c")
        ubase = wid * UNITS_W
        pltpu.async_copy(
            xt_hbm.at[pl.ds(ubase * IB, UNITS_W * IB)], idx_v, isem
        ).wait()

        iota = lax.iota(jnp.int32, L)
        iota128 = iota * IB

        def gather_blk(g, b):
            pltpu.async_copy(
                table_hbm.at[idx_v.at[pl.ds(g * CHUNK, CHUNK)]],
                rows_v.at[pl.ds(b * CHUNK, CHUNK)],
                gsems[b],
            )

        def gather_blk_wait(g, b):
            pltpu.make_async_copy(
                table_hbm.at[idx_v.at[pl.ds(g * CHUNK, CHUNK)]],
                rows_v.at[pl.ds(b * CHUNK, CHUNK)],
                gsems[b],
            ).wait()

        def store_unit(g, b, uu):
            u = ubase + g * UPB + uu
            h = u // NI
            i1 = u % NI
            t = b * UPB + uu
            for e1 in range(EMB_DIM // 8):
                pltpu.async_copy(
                    tout_v.at[pl.ds(t * TILE + e1 * 1024, 1024)],
                    out_hbm.at[h, e1, pl.ds(i1 * 1024, 1024)],
                    osems[t],
                )

        def store_unit_wait(g, b, uu):
            u = ubase + g * UPB + uu
            h = u // NI
            i1 = u % NI
            t = b * UPB + uu
            for e1 in range(EMB_DIM // 8):
                pltpu.make_async_copy(
                    tout_v.at[pl.ds(t * TILE + e1 * 1024, 1024)],
                    out_hbm.at[h, e1, pl.ds(i1 * 1024, 1024)],
                    osems[t],
                ).wait()

        def transpose_blk(b):
            row0 = b * CHUNK
            t0 = b * UPB
            base_a = iota128 + t0 * TILE
            base_b = base_a + L * IB
            base_c = iota128 + (t0 + 1) * TILE
            base_d = base_c + L * IB

            @plsc.parallel_loop(0, IB, unroll=8)
            def _(i2):
                va = rows_v[row0 + i2, pl.ds(0, L)]
                vb = rows_v[row0 + i2, pl.ds(L, L)]
                vc = rows_v[row0 + IB + i2, pl.ds(0, L)]
                vd = rows_v[row0 + IB + i2, pl.ds(L, L)]
                plsc.store_scatter(tout_v, [base_a + i2], va)
                plsc.store_scatter(tout_v, [base_b + i2], vb)
                plsc.store_scatter(tout_v, [base_c + i2], vc)
                plsc.store_scatter(tout_v, [base_d + i2], vd)

        for b in range(NBUF):
            gather_blk(b, b)

        def wave(i, carry):
            for b in range(NBUF):
                g = i * NBUF + b
                gather_blk_wait(g, b)
                @pl.when(i > 0)
                def _():
                    for uu in range(UPB):
                        store_unit_wait(g - NBUF, b, uu)
                transpose_blk(b)
                for uu in range(UPB):
                    store_unit(g, b, uu)
                @pl.when(i < NWAVE - 1)
                def _():
                    gather_blk(g + NBUF, b)
            return carry

        lax.fori_loop(0, NWAVE, wave, 0)

        for b in range(NBUF):
            for uu in range(UPB):
                store_unit_wait(NBLK - NBUF + b, b, uu)

    return gather_kernel


_gather = _make_kernel()


def kernel(x, table):
    xt_flat = x.T.reshape(BATCH * HIST).astype(jnp.int32)
    out3 = _gather(xt_flat, table)
    out5 = out3.reshape(HIST, EMB_DIM // 8, NI, 8, IB)
    return out5.transpose(2, 4, 0, 1, 3).reshape(BATCH, HIST, EMB_DIM)
